# baseline (speedup 1.0000x reference)
import functools

import jax
import jax.numpy as jnp
from jax.experimental import pallas as pl
from jax.experimental.pallas import tpu as pltpu


def _trig_tables(H, W):
    fov_up = 3.0 * jnp.pi / 180.0
    fov_down = -25.0 * jnp.pi / 180.0
    yaw = -jnp.pi + (jnp.arange(W, dtype=jnp.float32) + 0.5) / W * (2.0 * jnp.pi)
    pitch = fov_up - (jnp.arange(H, dtype=jnp.float32) + 0.5) / H * (fov_up - fov_down)
    cpv = jnp.broadcast_to(jnp.cos(pitch)[:, None], (H, W)).reshape(-1)
    spv = jnp.broadcast_to(jnp.sin(pitch)[:, None], (H, W)).reshape(-1)
    cyv = jnp.broadcast_to(jnp.cos(yaw)[None, :], (H, W)).reshape(-1)
    syv = jnp.broadcast_to(jnp.sin(yaw)[None, :], (H, W)).reshape(-1)
    return cpv, spv, cyv, syv


def _operands(image_pred, image_gt, cpv, spv, cyv, syv):
    B = image_pred.shape[0]
    rp = image_pred.reshape(B, -1)
    rg = image_gt.reshape(B, -1)
    rcp = rp * cpv
    rcg = rg * cpv
    px, py, pz = rcp * cyv, rcp * syv, rp * spv
    gx, gy, gz = rcg * cyv, rcg * syv, rg * spv
    p2 = px * px + py * py + pz * pz
    g2 = gx * gx + gy * gy + gz * gz
    lhs = jnp.stack([-2.0 * px, -2.0 * py, -2.0 * pz], axis=-1).astype(jnp.bfloat16)
    rhs = jnp.stack([gx, gy, gz], axis=1).astype(jnp.bfloat16)
    return lhs, rhs, p2[..., None], g2[:, None, :]


def _chamfer_kernel(p_ref, gt_ref, p2_ref, g2_ref, out_ref, colmin_ref,
                    *, n_rows, n_pts, n_batch):
    b = pl.program_id(0)
    n_rblocks = n_pts // n_rows
    gt = gt_ref[0]
    g2 = g2_ref[0]

    @pl.when(b == 0)
    def _():
        out_ref[...] = jnp.zeros((1, 1), jnp.float32)

    colmin_ref[...] = jnp.full((1, n_pts), jnp.inf, jnp.float32)

    def body(r, rowsum):
        p = p_ref[0, pl.ds(r * n_rows, n_rows), :]
        p2 = p2_ref[0, pl.ds(r * n_rows, n_rows), :]
        mmneg2 = jax.lax.dot_general(
            p, gt, (((1,), (0,)), ((), ())),
            preferred_element_type=jnp.float32,
        )
        d2 = (p2 + g2) + mmneg2
        rowmin = jnp.maximum(jnp.min(d2, axis=1), 0.0)
        blockmin = jnp.min(d2, axis=0, keepdims=True)
        colmin_ref[...] = jnp.minimum(colmin_ref[...], blockmin)
        return rowsum + jnp.sum(rowmin)

    rowsum = jax.lax.fori_loop(0, n_rblocks, body, jnp.float32(0.0), unroll=2)
    colsum = jnp.sum(jnp.maximum(colmin_ref[...], 0.0))
    scale = 1.0 / (n_pts * n_batch)
    out_ref[...] += (rowsum + colsum).reshape(1, 1) * scale


def _run_pallas(lhs, rhs, p2, g2, n_batch_total):
    B_local, N = lhs.shape[0], lhs.shape[1]
    ROWS = 512
    return pl.pallas_call(
        functools.partial(_chamfer_kernel, n_rows=ROWS, n_pts=N,
                          n_batch=n_batch_total),
        grid=(B_local,),
        in_specs=[
            pl.BlockSpec((1, N, 3), lambda b: (b, 0, 0)),
            pl.BlockSpec((1, 3, N), lambda b: (b, 0, 0)),
            pl.BlockSpec((1, N, 1), lambda b: (b, 0, 0)),
            pl.BlockSpec((1, 1, N), lambda b: (b, 0, 0)),
        ],
        out_specs=pl.BlockSpec((1, 1), lambda b: (0, 0)),
        out_shape=jax.ShapeDtypeStruct((1, 1), jnp.float32),
        scratch_shapes=[pltpu.VMEM((1, N), jnp.float32)],
        compiler_params=pltpu.CompilerParams(
            dimension_semantics=("arbitrary",),
        ),
    )(lhs, rhs, p2, g2)


@jax.jit
def kernel(image_pred, image_gt):
    B, H, W = image_pred.shape
    N = H * W
    cpv, spv, cyv, syv = _trig_tables(H, W)
    lhs, rhs, p2, g2 = _operands(image_pred, image_gt, cpv, spv, cyv, syv)

    out = _run_pallas(lhs, rhs, p2, g2, B)
    return out[0, 0]


# speedup vs baseline: 1.0294x; 1.0294x over previous
import functools

import jax
import jax.numpy as jnp
from jax.experimental import pallas as pl
from jax.experimental.pallas import tpu as pltpu


def _trig_tables(H, W):
    fov_up = 3.0 * jnp.pi / 180.0
    fov_down = -25.0 * jnp.pi / 180.0
    yaw = -jnp.pi + (jnp.arange(W, dtype=jnp.float32) + 0.5) / W * (2.0 * jnp.pi)
    pitch = fov_up - (jnp.arange(H, dtype=jnp.float32) + 0.5) / H * (fov_up - fov_down)
    cpv = jnp.broadcast_to(jnp.cos(pitch)[:, None], (H, W)).reshape(-1)
    spv = jnp.broadcast_to(jnp.sin(pitch)[:, None], (H, W)).reshape(-1)
    cyv = jnp.broadcast_to(jnp.cos(yaw)[None, :], (H, W)).reshape(-1)
    syv = jnp.broadcast_to(jnp.sin(yaw)[None, :], (H, W)).reshape(-1)
    return cpv, spv, cyv, syv


def _operands(image_pred, image_gt, cpv, spv, cyv, syv):
    B = image_pred.shape[0]
    rp = image_pred.reshape(B, -1)
    rg = image_gt.reshape(B, -1)
    rcp = rp * cpv
    rcg = rg * cpv
    px, py, pz = rcp * cyv, rcp * syv, rp * spv
    gx, gy, gz = rcg * cyv, rcg * syv, rg * spv
    p2 = px * px + py * py + pz * pz
    g2 = gx * gx + gy * gy + gz * gz
    lhs = jnp.stack([-2.0 * px, -2.0 * py, -2.0 * pz], axis=-1).astype(jnp.bfloat16)
    rhs = jnp.stack([gx, gy, gz], axis=1).astype(jnp.bfloat16)
    return lhs, rhs, p2[..., None], g2[:, None, :]


def _chamfer_kernel(p_ref, gt_ref, p2_ref, g2_ref, out_ref, colmin_ref,
                    *, n_rows, n_pts, n_batch):
    b = pl.program_id(0)
    n_rblocks = n_pts // n_rows
    gt = gt_ref[0]
    g2 = g2_ref[0]

    @pl.when(b == 0)
    def _():
        out_ref[...] = jnp.zeros((1, 1), jnp.float32)

    colmin_ref[...] = jnp.full((1, n_pts), jnp.inf, jnp.float32)

    def body(r, rowsum):
        p = p_ref[0, pl.ds(r * n_rows, n_rows), :]
        p2 = p2_ref[0, pl.ds(r * n_rows, n_rows), :]
        mmneg2 = jax.lax.dot_general(
            p, gt, (((1,), (0,)), ((), ())),
            preferred_element_type=jnp.float32,
        )
        d2 = (p2 + g2) + mmneg2
        rowmin = jnp.maximum(jnp.min(d2, axis=1), 0.0)
        blockmin = jnp.min(d2, axis=0, keepdims=True)
        colmin_ref[...] = jnp.minimum(colmin_ref[...], blockmin)
        return rowsum + jnp.sum(rowmin)

    rowsum = jax.lax.fori_loop(0, n_rblocks, body, jnp.float32(0.0), unroll=2)
    colsum = jnp.sum(jnp.maximum(colmin_ref[...], 0.0))
    scale = 1.0 / (n_pts * n_batch)
    out_ref[...] += (rowsum + colsum).reshape(1, 1) * scale


def _run_pallas(lhs, rhs, p2, g2, n_batch_total):
    B_local, N = lhs.shape[0], lhs.shape[1]
    ROWS = 1024
    return pl.pallas_call(
        functools.partial(_chamfer_kernel, n_rows=ROWS, n_pts=N,
                          n_batch=n_batch_total),
        grid=(B_local,),
        in_specs=[
            pl.BlockSpec((1, N, 3), lambda b: (b, 0, 0)),
            pl.BlockSpec((1, 3, N), lambda b: (b, 0, 0)),
            pl.BlockSpec((1, N, 1), lambda b: (b, 0, 0)),
            pl.BlockSpec((1, 1, N), lambda b: (b, 0, 0)),
        ],
        out_specs=pl.BlockSpec((1, 1), lambda b: (0, 0)),
        out_shape=jax.ShapeDtypeStruct((1, 1), jnp.float32),
        scratch_shapes=[pltpu.VMEM((1, N), jnp.float32)],
        compiler_params=pltpu.CompilerParams(
            dimension_semantics=("arbitrary",),
        ),
    )(lhs, rhs, p2, g2)


@jax.jit
def kernel(image_pred, image_gt):
    B, H, W = image_pred.shape
    N = H * W
    cpv, spv, cyv, syv = _trig_tables(H, W)
    lhs, rhs, p2, g2 = _operands(image_pred, image_gt, cpv, spv, cyv, syv)

    out = _run_pallas(lhs, rhs, p2, g2, B)
    return out[0, 0]


# speedup vs baseline: 1.0325x; 1.0030x over previous
import functools

import jax
import jax.numpy as jnp
from jax.experimental import pallas as pl
from jax.experimental.pallas import tpu as pltpu


def _trig_tables(H, W):
    fov_up = 3.0 * jnp.pi / 180.0
    fov_down = -25.0 * jnp.pi / 180.0
    yaw = -jnp.pi + (jnp.arange(W, dtype=jnp.float32) + 0.5) / W * (2.0 * jnp.pi)
    pitch = fov_up - (jnp.arange(H, dtype=jnp.float32) + 0.5) / H * (fov_up - fov_down)
    cpv = jnp.broadcast_to(jnp.cos(pitch)[:, None], (H, W)).reshape(-1)
    spv = jnp.broadcast_to(jnp.sin(pitch)[:, None], (H, W)).reshape(-1)
    cyv = jnp.broadcast_to(jnp.cos(yaw)[None, :], (H, W)).reshape(-1)
    syv = jnp.broadcast_to(jnp.sin(yaw)[None, :], (H, W)).reshape(-1)
    return cpv, spv, cyv, syv


def _operands(image_pred, image_gt, cpv, spv, cyv, syv):
    B = image_pred.shape[0]
    rp = image_pred.reshape(B, -1)
    rg = image_gt.reshape(B, -1)
    rcp = rp * cpv
    rcg = rg * cpv
    px, py, pz = rcp * cyv, rcp * syv, rp * spv
    gx, gy, gz = rcg * cyv, rcg * syv, rg * spv
    p2 = px * px + py * py + pz * pz
    g2 = gx * gx + gy * gy + gz * gz
    lhs = jnp.stack([-2.0 * px, -2.0 * py, -2.0 * pz], axis=-1).astype(jnp.bfloat16)
    rhs = jnp.stack([gx, gy, gz], axis=1).astype(jnp.bfloat16)
    return lhs, rhs, p2[..., None], g2[:, None, :]


def _chamfer_kernel(p_ref, gt_ref, p2_ref, g2_ref, out_ref, colmin_ref,
                    *, n_rows, n_pts, n_batch):
    b = pl.program_id(0)
    n_rblocks = n_pts // n_rows
    gt = gt_ref[0]
    g2 = g2_ref[0]

    @pl.when(b == 0)
    def _():
        out_ref[...] = jnp.zeros((1, 1), jnp.float32)

    colmin_ref[...] = jnp.full((1, n_pts), jnp.inf, jnp.float32)

    def body(r, rowsum):
        p = p_ref[0, pl.ds(r * n_rows, n_rows), :]
        p2 = p2_ref[0, pl.ds(r * n_rows, n_rows), :]
        mmneg2 = jax.lax.dot_general(
            p, gt, (((1,), (0,)), ((), ())),
            preferred_element_type=jnp.float32,
        )
        d2 = (p2 + g2) + mmneg2
        rowmin = jnp.maximum(jnp.min(d2, axis=1), 0.0)
        blockmin = jnp.min(d2, axis=0, keepdims=True)
        colmin_ref[...] = jnp.minimum(colmin_ref[...], blockmin)
        return rowsum + jnp.sum(rowmin)

    rowsum = jax.lax.fori_loop(0, n_rblocks, body, jnp.float32(0.0), unroll=4)
    colsum = jnp.sum(jnp.maximum(colmin_ref[...], 0.0))
    scale = 1.0 / (n_pts * n_batch)
    out_ref[...] += (rowsum + colsum).reshape(1, 1) * scale


def _run_pallas(lhs, rhs, p2, g2, n_batch_total):
    B_local, N = lhs.shape[0], lhs.shape[1]
    ROWS = 1024
    return pl.pallas_call(
        functools.partial(_chamfer_kernel, n_rows=ROWS, n_pts=N,
                          n_batch=n_batch_total),
        grid=(B_local,),
        in_specs=[
            pl.BlockSpec((1, N, 3), lambda b: (b, 0, 0)),
            pl.BlockSpec((1, 3, N), lambda b: (b, 0, 0)),
            pl.BlockSpec((1, N, 1), lambda b: (b, 0, 0)),
            pl.BlockSpec((1, 1, N), lambda b: (b, 0, 0)),
        ],
        out_specs=pl.BlockSpec((1, 1), lambda b: (0, 0)),
        out_shape=jax.ShapeDtypeStruct((1, 1), jnp.float32),
        scratch_shapes=[pltpu.VMEM((1, N), jnp.float32)],
        compiler_params=pltpu.CompilerParams(
            dimension_semantics=("arbitrary",),
        ),
    )(lhs, rhs, p2, g2)


@jax.jit
def kernel(image_pred, image_gt):
    B, H, W = image_pred.shape
    N = H * W
    cpv, spv, cyv, syv = _trig_tables(H, W)
    lhs, rhs, p2, g2 = _operands(image_pred, image_gt, cpv, spv, cyv, syv)

    out = _run_pallas(lhs, rhs, p2, g2, B)
    return out[0, 0]
